# baseline (speedup 1.0000x reference)
import jax
import jax.numpy as jnp
from jax import lax
from jax.experimental import pallas as pl
from jax.experimental.pallas import tpu as pltpu

BATCH = 16384
DIM = 64
T = 128
NSTEP = BATCH // T


def _tc_body(s_sm, r_sm, d_sm, emb_hbm, rel_hbm, out_ref,
             hrows, rrows, trows, sem, sem2, sem3):
    step = pl.program_id(0)
    base = step * T

    for k in range(T):
        pltpu.make_async_copy(
            emb_hbm.at[pl.ds(s_sm[base + k], 1)], hrows.at[pl.ds(k, 1)], sem
        ).start()
        pltpu.make_async_copy(
            rel_hbm.at[pl.ds(r_sm[base + k], 1)], rrows.at[pl.ds(k, 1)], sem2
        ).start()
        pltpu.make_async_copy(
            emb_hbm.at[pl.ds(d_sm[base + k], 1)], trows.at[pl.ds(k, 1)], sem3
        ).start()

    pltpu.make_async_copy(emb_hbm.at[pl.ds(0, T)], hrows, sem).wait()
    pltpu.make_async_copy(emb_hbm.at[pl.ds(0, T)], rrows, sem2).wait()
    pltpu.make_async_copy(emb_hbm.at[pl.ds(0, T)], trows, sem3).wait()

    score = jnp.sum(jnp.abs(hrows[...] + rrows[...] - trows[...]), axis=1)
    out_ref[...] = 1.0 / (1.0 + jnp.exp(score))


def kernel(x, emb, rel_emb):
    xi = x.astype(jnp.int32)
    grid_spec = pltpu.PrefetchScalarGridSpec(
        num_scalar_prefetch=3,
        grid=(NSTEP,),
        in_specs=[
            pl.BlockSpec(memory_space=pltpu.MemorySpace.HBM),
            pl.BlockSpec(memory_space=pltpu.MemorySpace.HBM),
        ],
        out_specs=pl.BlockSpec((T,), lambda i, *_: (i,)),
        scratch_shapes=[
            pltpu.VMEM((T, DIM), jnp.float32),
            pltpu.VMEM((T, DIM), jnp.float32),
            pltpu.VMEM((T, DIM), jnp.float32),
            pltpu.SemaphoreType.DMA,
            pltpu.SemaphoreType.DMA,
            pltpu.SemaphoreType.DMA,
        ],
    )
    run = pl.pallas_call(
        _tc_body,
        grid_spec=grid_spec,
        out_shape=jax.ShapeDtypeStruct((BATCH,), jnp.float32),
    )
    return run(xi[:, 0], xi[:, 1], xi[:, 2], emb, rel_emb)


# speedup vs baseline: 1.2221x; 1.2221x over previous
import jax
import jax.numpy as jnp
from jax import lax
from jax.experimental import pallas as pl
from jax.experimental.pallas import tpu as pltpu
from jax.experimental.pallas import tpu_sc as plsc

BATCH = 16384
DIM = 64
LANES = 16
NUM_WORKERS = 32
NSC = 10240
NTC = BATCH - NSC
BPW = NSC // NUM_WORKERS
G = 64
NST = BPW // G
T = 128
NSTEP = NTC // T


def _sc_body(s_hbm, r_hbm, d_hbm, emb_hbm, rel_hbm, out_hbm,
             sidx, ridx, didx, hbuf, rbuf, tbuf, outv, sem, sem2, sem3):
    cid = lax.axis_index("c")
    sid = lax.axis_index("s")
    wid = sid * 2 + cid
    base = wid * BPW

    lanes = lax.iota(jnp.int32, LANES)

    pltpu.sync_copy(s_hbm.at[pl.ds(base, BPW)], sidx)
    pltpu.sync_copy(r_hbm.at[pl.ds(base, BPW)], ridx)
    pltpu.sync_copy(d_hbm.at[pl.ds(base, BPW)], didx)

    def fire(stage, b):
        def fire_grp(g, carry):
            off = stage * G + g * LANES
            vs = sidx[pl.ds(off, LANES)]
            vr = ridx[pl.ds(off, LANES)]
            vd = didx[pl.ds(off, LANES)]
            for k in range(LANES):
                row = pl.ds((g * LANES + k) * DIM, DIM)
                pltpu.async_copy(emb_hbm.at[vs[k]], hbuf.at[b, row], sem)
                pltpu.async_copy(rel_hbm.at[vr[k]], rbuf.at[b, row], sem2)
                pltpu.async_copy(emb_hbm.at[vd[k]], tbuf.at[b, row], sem3)
            return carry

        lax.fori_loop(0, G // LANES, fire_grp, 0)

    def drain(b):
        pltpu.make_async_copy(out_hbm.at[pl.ds(0, G * DIM)], hbuf.at[b], sem).wait()
        pltpu.make_async_copy(out_hbm.at[pl.ds(0, G * DIM)], rbuf.at[b], sem2).wait()
        pltpu.make_async_copy(out_hbm.at[pl.ds(0, G * DIM)], tbuf.at[b], sem3).wait()

    def compute(stage, b):
        def cg(g, carry):
            acc = jnp.zeros((LANES,), jnp.float32)
            for k in range(LANES):
                row = g * LANES + k
                w = jnp.zeros((LANES,), jnp.float32)
                for j in range(DIM // LANES):
                    sl = pl.ds(row * DIM + j * LANES, LANES)
                    w = w + jnp.abs(hbuf[b, sl] + rbuf[b, sl] - tbuf[b, sl])
                acc = jnp.where(lanes == k, jnp.sum(w), acc)
            outv[pl.ds(stage * G + g * LANES, LANES)] = 1.0 / (1.0 + jnp.exp(acc))
            return carry

        lax.fori_loop(0, G // LANES, cg, 0)

    fire(0, 0)

    def stage_loop(s, carry):
        b = lax.rem(s, 2)
        drain(b)

        @pl.when(s < NST - 1)
        def _():
            fire(s + 1, 1 - b)

        compute(s, b)
        return carry

    lax.fori_loop(0, NST, stage_loop, 0)

    pltpu.sync_copy(outv, out_hbm.at[pl.ds(base, BPW)])


def _tc_body(s_sm, r_sm, d_sm, emb_hbm, rel_hbm, out_ref,
             hrows, rrows, trows, sem, sem2, sem3):
    step = pl.program_id(0)
    base = step * T

    for k in range(T):
        pltpu.make_async_copy(
            emb_hbm.at[pl.ds(s_sm[base + k], 1)], hrows.at[pl.ds(k, 1)], sem
        ).start()
        pltpu.make_async_copy(
            rel_hbm.at[pl.ds(r_sm[base + k], 1)], rrows.at[pl.ds(k, 1)], sem2
        ).start()
        pltpu.make_async_copy(
            emb_hbm.at[pl.ds(d_sm[base + k], 1)], trows.at[pl.ds(k, 1)], sem3
        ).start()

    pltpu.make_async_copy(emb_hbm.at[pl.ds(0, T)], hrows, sem).wait()
    pltpu.make_async_copy(emb_hbm.at[pl.ds(0, T)], rrows, sem2).wait()
    pltpu.make_async_copy(emb_hbm.at[pl.ds(0, T)], trows, sem3).wait()

    score = jnp.sum(jnp.abs(hrows[...] + rrows[...] - trows[...]), axis=1)
    out_ref[...] = 1.0 / (1.0 + jnp.exp(score))


def _sc_call(s, r, d, emb, rel_emb):
    mesh = plsc.VectorSubcoreMesh(core_axis_name="c", subcore_axis_name="s")
    run = pl.kernel(
        _sc_body,
        out_type=jax.ShapeDtypeStruct((NSC,), jnp.float32),
        mesh=mesh,
        compiler_params=pltpu.CompilerParams(needs_layout_passes=False),
        scratch_types=[
            pltpu.VMEM((BPW,), jnp.int32),
            pltpu.VMEM((BPW,), jnp.int32),
            pltpu.VMEM((BPW,), jnp.int32),
            pltpu.VMEM((2, G * DIM), jnp.float32),
            pltpu.VMEM((2, G * DIM), jnp.float32),
            pltpu.VMEM((2, G * DIM), jnp.float32),
            pltpu.VMEM((BPW,), jnp.float32),
            pltpu.SemaphoreType.DMA,
            pltpu.SemaphoreType.DMA,
            pltpu.SemaphoreType.DMA,
        ],
    )
    return run(s, r, d, emb, rel_emb)


def _tc_call(s, r, d, emb, rel_emb):
    grid_spec = pltpu.PrefetchScalarGridSpec(
        num_scalar_prefetch=3,
        grid=(NSTEP,),
        in_specs=[
            pl.BlockSpec(memory_space=pltpu.MemorySpace.HBM),
            pl.BlockSpec(memory_space=pltpu.MemorySpace.HBM),
        ],
        out_specs=pl.BlockSpec((T,), lambda i, *_: (i,)),
        scratch_shapes=[
            pltpu.VMEM((T, DIM), jnp.float32),
            pltpu.VMEM((T, DIM), jnp.float32),
            pltpu.VMEM((T, DIM), jnp.float32),
            pltpu.SemaphoreType.DMA,
            pltpu.SemaphoreType.DMA,
            pltpu.SemaphoreType.DMA,
        ],
    )
    run = pl.pallas_call(
        _tc_body,
        grid_spec=grid_spec,
        out_shape=jax.ShapeDtypeStruct((NTC,), jnp.float32),
    )
    return run(s, r, d, emb, rel_emb)


def kernel(x, emb, rel_emb):
    xi = x.astype(jnp.int32)
    s, r, d = xi[:, 0], xi[:, 1], xi[:, 2]
    out_sc = _sc_call(s[:NSC], r[:NSC], d[:NSC], emb, rel_emb)
    out_tc = _tc_call(s[NSC:], r[NSC:], d[NSC:], emb, rel_emb)
    return jnp.concatenate([out_sc, out_tc])


# speedup vs baseline: 1.4113x; 1.1548x over previous
import jax
import jax.numpy as jnp
from jax import lax
from jax.experimental import pallas as pl
from jax.experimental.pallas import tpu as pltpu
from jax.experimental.pallas import tpu_sc as plsc

BATCH = 16384
DIM = 64
LANES = 16
NUM_WORKERS = 32
BPW = BATCH // NUM_WORKERS
G = 128
NST = BPW // G


def _body(s_hbm, r_hbm, d_hbm, emb_hbm, rel_hbm, out_hbm,
          sidx, ridx, didx, hbuf, rbuf, tbuf, outv, sem, sem2, sem3):
    cid = lax.axis_index("c")
    sid = lax.axis_index("s")
    wid = sid * 2 + cid
    base = wid * BPW

    lanes = lax.iota(jnp.int32, LANES)

    pltpu.sync_copy(s_hbm.at[pl.ds(base, BPW)], sidx)
    pltpu.sync_copy(r_hbm.at[pl.ds(base, BPW)], ridx)
    pltpu.sync_copy(d_hbm.at[pl.ds(base, BPW)], didx)

    def fire(stage, b):
        def fire_grp(g, carry):
            off = stage * G + g * LANES
            vs = sidx[pl.ds(off, LANES)]
            vr = ridx[pl.ds(off, LANES)]
            vd = didx[pl.ds(off, LANES)]
            for k in range(LANES):
                row = pl.ds((g * LANES + k) * DIM, DIM)
                pltpu.async_copy(emb_hbm.at[vs[k]], hbuf.at[b, row], sem)
                pltpu.async_copy(rel_hbm.at[vr[k]], rbuf.at[b, row], sem2)
                pltpu.async_copy(emb_hbm.at[vd[k]], tbuf.at[b, row], sem3)
            return carry

        lax.fori_loop(0, G // LANES, fire_grp, 0)

    def drain(b):
        pltpu.make_async_copy(out_hbm.at[pl.ds(0, G * DIM)], hbuf.at[b], sem).wait()
        pltpu.make_async_copy(out_hbm.at[pl.ds(0, G * DIM)], rbuf.at[b], sem2).wait()
        pltpu.make_async_copy(out_hbm.at[pl.ds(0, G * DIM)], tbuf.at[b], sem3).wait()


    def compute(stage, b):
        def cg(g, carry):
            acc = jnp.zeros((LANES,), jnp.float32)
            for k in range(LANES):
                row = g * LANES + k
                w = jnp.zeros((LANES,), jnp.float32)
                for j in range(DIM // LANES):
                    h = hbuf[b, pl.ds(row * DIM + j * LANES, LANES)]
                    rr = rbuf[b, pl.ds(row * DIM + j * LANES, LANES)]
                    t = tbuf[b, pl.ds(row * DIM + j * LANES, LANES)]
                    w = w + jnp.abs(h + rr - t)
                acc = jnp.where(lanes == k, jnp.sum(w), acc)
            outv[pl.ds(stage * G + g * LANES, LANES)] = 1.0 / (1.0 + jnp.exp(acc))
            return carry

        lax.fori_loop(0, G // LANES, cg, 0)

    fire(0, 0)

    def stage_loop(s, carry):
        b = lax.rem(s, 2)
        drain(b)

        @pl.when(s < NST - 1)
        def _():
            fire(s + 1, 1 - b)

        compute(s, b)
        return carry

    lax.fori_loop(0, NST, stage_loop, 0)

    pltpu.sync_copy(outv, out_hbm.at[pl.ds(base, BPW)])


def kernel(x, emb, rel_emb):
    mesh = plsc.VectorSubcoreMesh(core_axis_name="c", subcore_axis_name="s")
    run = pl.kernel(
        _body,
        out_type=jax.ShapeDtypeStruct((BATCH,), jnp.float32),
        mesh=mesh,
        compiler_params=pltpu.CompilerParams(needs_layout_passes=False),
        scratch_types=[
            pltpu.VMEM((BPW,), jnp.int32),
            pltpu.VMEM((BPW,), jnp.int32),
            pltpu.VMEM((BPW,), jnp.int32),
            pltpu.VMEM((2, G * DIM), jnp.float32),
            pltpu.VMEM((2, G * DIM), jnp.float32),
            pltpu.VMEM((2, G * DIM), jnp.float32),
            pltpu.VMEM((BPW,), jnp.float32),
            pltpu.SemaphoreType.DMA,
            pltpu.SemaphoreType.DMA,
            pltpu.SemaphoreType.DMA,
        ],
    )
    xi = x.astype(jnp.int32)
    return run(xi[:, 0], xi[:, 1], xi[:, 2], emb, rel_emb)
